# baseline (speedup 1.0000x reference)
import functools

import jax
import jax.numpy as jnp
import numpy as np
from jax import lax
from jax.experimental import pallas as pl
from jax.experimental.pallas import tpu as pltpu
from jax.experimental.pallas import tpu_sc as plsc

_NX, _NY, _NT, _NS = 512, 512, 2048, 256
_K = np.float32(1.0 / (1500.0 * 2e-8))
_TMAX_IDX = np.float32((_NT - 2) * 2e-8 / 2e-8)

_SBLK = 8
_IBLK = 128


def _idx_body(u2_ref, v2_ref, out_ref):
    u2 = u2_ref[...]
    v2 = v2_ref[...]
    d2 = u2[:, :, None] + v2[None, None, :]
    idx = jnp.sqrt(d2) * _K
    idx = jnp.where(idx > _TMAX_IDX, jnp.float32(0.0), idx)
    out_ref[...] = idx


def _compute_idx(u2, v2):
    ns = u2.shape[0]
    return pl.pallas_call(
        _idx_body,
        grid=(ns // _SBLK, _NX // _IBLK),
        in_specs=[
            pl.BlockSpec((_SBLK, _IBLK), lambda s, i: (s, i)),
            pl.BlockSpec((_NY,), lambda s, i: (0,)),
        ],
        out_specs=pl.BlockSpec((_SBLK, _IBLK, _NY), lambda s, i: (s, i, 0)),
        out_shape=jax.ShapeDtypeStruct((ns, _NX, _NY), jnp.float32),
    )(u2, v2)


def _das_sc(x, idxs, ns):
    info = plsc.get_sparse_core_info()
    nc, nsc = info.num_cores, info.num_subcores
    nw = nc * nsc
    px_w = _NX * _NY // nw
    nchunks = px_w // 16

    mesh = plsc.VectorSubcoreMesh(core_axis_name="c", subcore_axis_name="s")

    @functools.partial(
        pl.kernel,
        out_type=jax.ShapeDtypeStruct((nw, px_w), jnp.float32),
        mesh=mesh,
        scratch_types=[
            pltpu.VMEM((_NT,), jnp.float32),
            pltpu.VMEM((_NT,), jnp.float32),
            pltpu.VMEM((px_w,), jnp.float32),
            pltpu.VMEM((px_w,), jnp.float32),
            pltpu.VMEM((px_w,), jnp.float32),
            pltpu.SemaphoreType.DMA,
            pltpu.SemaphoreType.DMA,
            pltpu.SemaphoreType.DMA,
            pltpu.SemaphoreType.DMA,
        ],
        compiler_params=pltpu.CompilerParams(needs_layout_passes=False),
    )
    def k(x_hbm, idx_hbm, out_hbm, tr0, tr1, ix0, ix1, acc_v, st0, st1, si0, si1):
        wid = lax.axis_index("s") * nc + lax.axis_index("c")
        trace_b = [tr0, tr1]
        idx_b = [ix0, ix1]
        strc = [st0, st1]
        sidx = [si0, si1]

        def start(s, b):
            pltpu.async_copy(x_hbm.at[0, s], trace_b[b], strc[b])
            pltpu.async_copy(idx_hbm.at[s, wid], idx_b[b], sidx[b])

        def wait(s, b):
            pltpu.make_async_copy(x_hbm.at[0, s], trace_b[b], strc[b]).wait()
            pltpu.make_async_copy(idx_hbm.at[s, wid], idx_b[b], sidx[b]).wait()

        start(0, 0)

        zeros = jnp.zeros((16,), jnp.float32)

        @pl.loop(0, nchunks)
        def _z(c):
            acc_v[pl.ds(c * 16, 16)] = zeros

        @pl.loop(0, ns, step=2)
        def _sensor(s0):
            for b in range(2):
                s = s0 + b

                @pl.when(s + 1 < ns)
                def _prefetch():
                    start(s + 1, 1 - b)

                wait(s, b)
                trace = trace_b[b]
                idxv = idx_b[b]

                @pl.loop(0, nchunks, step=4)
                def _chunk(c):
                    for k in range(4):
                        sl = pl.ds((c + k) * 16, 16)
                        v = idxv[sl]
                        d0 = v.astype(jnp.int32)
                        w0 = v - d0.astype(jnp.float32)
                        y0 = plsc.load_gather(trace, [d0])
                        y1 = plsc.load_gather(trace, [d0 + 1])
                        plsc.addupdate(acc_v.at[sl], y1 + w0 * (y0 - y1))

        pltpu.sync_copy(acc_v, out_hbm.at[wid])

    return k(x, idxs)


def kernel(x, gridF, sensors):
    ax = gridF[:, 0, 0]
    ay = gridF[0, :, 1]
    sx = sensors[:, 0]
    du = ax[None, :] - sx[:, None]
    u2 = du * du
    dv = ay - sensors[0, 1]
    v2 = dv * dv
    ngrp = 4
    sg = _NS // ngrp
    parts = []
    for g in range(ngrp):
        u2_g = lax.slice_in_dim(u2, g * sg, (g + 1) * sg, axis=0)
        x_g = lax.slice_in_dim(x, g * sg, (g + 1) * sg, axis=1)
        idx_g = _compute_idx(u2_g, v2)
        idx_g = idx_g.reshape(sg, 32, _NX * _NY // 32)
        parts.append(_das_sc(x_g, idx_g, sg))
    out = parts[0] + parts[1] + parts[2] + parts[3]
    return out.reshape(1, _NX, _NY)


# speedup vs baseline: 1.2277x; 1.2277x over previous
import functools

import jax
import jax.numpy as jnp
import numpy as np
from jax import lax
from jax.experimental import pallas as pl
from jax.experimental.pallas import tpu as pltpu
from jax.experimental.pallas import tpu_sc as plsc

_NX, _NY, _NT, _NS = 512, 512, 2048, 256
_K = np.float32(1.0 / (1500.0 * 2e-8))
_TMAX_IDX = np.float32((_NT - 2) * 2e-8 / 2e-8)

_SBLK = 8
_IBLK = 128


def _idx_body(u2_ref, v2_ref, out_ref):
    u2 = u2_ref[...]
    v2 = v2_ref[...]
    d2 = u2[:, :, None] + v2[None, None, :]
    idx = jnp.sqrt(d2) * _K
    idx = jnp.where(idx > _TMAX_IDX, jnp.float32(0.0), idx)
    out_ref[...] = idx


def _compute_idx(u2, v2):
    ns = u2.shape[0]
    return pl.pallas_call(
        _idx_body,
        grid=(ns // _SBLK, _NX // _IBLK),
        in_specs=[
            pl.BlockSpec((_SBLK, _IBLK), lambda s, i: (s, i)),
            pl.BlockSpec((_NY,), lambda s, i: (0,)),
        ],
        out_specs=pl.BlockSpec((_SBLK, _IBLK, _NY), lambda s, i: (s, i, 0)),
        out_shape=jax.ShapeDtypeStruct((ns, _NX, _NY), jnp.float32),
    )(u2, v2)


def _das_sc(x, idxs, ns):
    info = plsc.get_sparse_core_info()
    nc, nsc = info.num_cores, info.num_subcores
    nw = nc * nsc
    px_w = _NX * _NY // nw
    nchunks = px_w // 16

    mesh = plsc.VectorSubcoreMesh(core_axis_name="c", subcore_axis_name="s")

    @functools.partial(
        pl.kernel,
        out_type=jax.ShapeDtypeStruct((nw, px_w), jnp.float32),
        mesh=mesh,
        scratch_types=[
            pltpu.VMEM((_NT,), jnp.float32),
            pltpu.VMEM((_NT,), jnp.float32),
            pltpu.VMEM((px_w,), jnp.float32),
            pltpu.VMEM((px_w,), jnp.float32),
            pltpu.VMEM((px_w,), jnp.float32),
            pltpu.SemaphoreType.DMA,
            pltpu.SemaphoreType.DMA,
            pltpu.SemaphoreType.DMA,
            pltpu.SemaphoreType.DMA,
        ],
        compiler_params=pltpu.CompilerParams(needs_layout_passes=False),
    )
    def k(x_hbm, idx_hbm, out_hbm, tr0, tr1, ix0, ix1, acc_v, st0, st1, si0, si1):
        wid = lax.axis_index("s") * nc + lax.axis_index("c")
        trace_b = [tr0, tr1]
        idx_b = [ix0, ix1]
        strc = [st0, st1]
        sidx = [si0, si1]

        def start(s, b):
            pltpu.async_copy(x_hbm.at[0, s], trace_b[b], strc[b])
            pltpu.async_copy(idx_hbm.at[s, wid], idx_b[b], sidx[b])

        def wait(s, b):
            pltpu.make_async_copy(x_hbm.at[0, s], trace_b[b], strc[b]).wait()
            pltpu.make_async_copy(idx_hbm.at[s, wid], idx_b[b], sidx[b]).wait()

        start(0, 0)

        zeros = jnp.zeros((16,), jnp.float32)

        @pl.loop(0, nchunks)
        def _z(c):
            acc_v[pl.ds(c * 16, 16)] = zeros

        @pl.loop(0, ns, step=2)
        def _sensor(s0):
            for b in range(2):
                s = s0 + b

                @pl.when(s + 1 < ns)
                def _prefetch():
                    start(s + 1, 1 - b)

                wait(s, b)
                trace = trace_b[b]
                idxv = idx_b[b]

                @pl.loop(0, nchunks, step=8)
                def _chunk(c):
                    for k in range(8):
                        sl = pl.ds((c + k) * 16, 16)
                        v = idxv[sl]
                        d0 = v.astype(jnp.int32)
                        d0f = d0.astype(jnp.float32)
                        w0 = v - d0f
                        w1 = jnp.float32(1.0) - w0
                        y0 = plsc.load_gather(trace, [d0])
                        y1 = plsc.load_gather(trace, [d0 + 1])
                        plsc.addupdate(acc_v.at[sl], w0 * y0 + w1 * y1)

        pltpu.sync_copy(acc_v, out_hbm.at[wid])

    return k(x, idxs)


def kernel(x, gridF, sensors):
    ax = gridF[:, 0, 0]
    ay = gridF[0, :, 1]
    sx = sensors[:, 0]
    du = ax[None, :] - sx[:, None]
    u2 = du * du
    dv = ay - sensors[0, 1]
    v2 = dv * dv
    ngrp = 4
    sg = _NS // ngrp
    parts = []
    for g in range(ngrp):
        u2_g = lax.slice_in_dim(u2, g * sg, (g + 1) * sg, axis=0)
        x_g = lax.slice_in_dim(x, g * sg, (g + 1) * sg, axis=1)
        idx_g = _compute_idx(u2_g, v2)
        idx_g = idx_g.reshape(sg, 32, _NX * _NY // 32)
        parts.append(_das_sc(x_g, idx_g, sg))
    out = parts[0] + parts[1] + parts[2] + parts[3]
    return out.reshape(1, _NX, _NY)


# speedup vs baseline: 1.2470x; 1.0157x over previous
import functools

import jax
import jax.numpy as jnp
import numpy as np
from jax import lax
from jax.experimental import pallas as pl
from jax.experimental.pallas import tpu as pltpu
from jax.experimental.pallas import tpu_sc as plsc

_NX, _NY, _NT, _NS = 512, 512, 2048, 256
_K = np.float32(1.0 / (1500.0 * 2e-8))
_TMAX_IDX = np.float32((_NT - 2) * 2e-8 / 2e-8)

_SBLK = 8
_IBLK = 128


def _idx_body(u2_ref, v2_ref, out_ref):
    u2 = u2_ref[...]
    v2 = v2_ref[...]
    d2 = u2[:, :, None] + v2[None, None, :]
    idx = jnp.sqrt(d2) * _K
    idx = jnp.where(idx > _TMAX_IDX, jnp.float32(0.0), idx)
    out_ref[...] = idx


def _compute_idx(u2, v2):
    ns = u2.shape[0]
    return pl.pallas_call(
        _idx_body,
        grid=(ns // _SBLK, _NX // _IBLK),
        in_specs=[
            pl.BlockSpec((_SBLK, _IBLK), lambda s, i: (s, i)),
            pl.BlockSpec((_NY,), lambda s, i: (0,)),
        ],
        out_specs=pl.BlockSpec((_SBLK, _IBLK, _NY), lambda s, i: (s, i, 0)),
        out_shape=jax.ShapeDtypeStruct((ns, _NX, _NY), jnp.float32),
    )(u2, v2)


def _das_sc(x, idxs, ns):
    info = plsc.get_sparse_core_info()
    nc, nsc = info.num_cores, info.num_subcores
    nw = nc * nsc
    px_w = _NX * _NY // nw
    nchunks = px_w // 16

    mesh = plsc.VectorSubcoreMesh(core_axis_name="c", subcore_axis_name="s")

    @functools.partial(
        pl.kernel,
        out_type=jax.ShapeDtypeStruct((nw, px_w), jnp.float32),
        mesh=mesh,
        scratch_types=[
            pltpu.VMEM((_NT,), jnp.float32),
            pltpu.VMEM((_NT,), jnp.float32),
            pltpu.VMEM((px_w,), jnp.float32),
            pltpu.VMEM((px_w,), jnp.float32),
            pltpu.VMEM((px_w,), jnp.float32),
            pltpu.SemaphoreType.DMA,
            pltpu.SemaphoreType.DMA,
            pltpu.SemaphoreType.DMA,
            pltpu.SemaphoreType.DMA,
        ],
        compiler_params=pltpu.CompilerParams(needs_layout_passes=False),
    )
    def k(x_hbm, idx_hbm, out_hbm, tr0, tr1, ix0, ix1, acc_v, st0, st1, si0, si1):
        wid = lax.axis_index("s") * nc + lax.axis_index("c")
        trace_b = [tr0, tr1]
        idx_b = [ix0, ix1]
        strc = [st0, st1]
        sidx = [si0, si1]

        def start(s, b):
            pltpu.async_copy(x_hbm.at[0, s], trace_b[b], strc[b])
            pltpu.async_copy(idx_hbm.at[s, wid], idx_b[b], sidx[b])

        def wait(s, b):
            pltpu.make_async_copy(x_hbm.at[0, s], trace_b[b], strc[b]).wait()
            pltpu.make_async_copy(idx_hbm.at[s, wid], idx_b[b], sidx[b]).wait()

        start(0, 0)

        zeros = jnp.zeros((16,), jnp.float32)

        @pl.loop(0, nchunks)
        def _z(c):
            acc_v[pl.ds(c * 16, 16)] = zeros

        @pl.loop(0, ns, step=2)
        def _sensor(s0):
            for b in range(2):
                s = s0 + b

                @pl.when(s + 1 < ns)
                def _prefetch():
                    start(s + 1, 1 - b)

                wait(s, b)
                trace = trace_b[b]
                idxv = idx_b[b]

                @pl.loop(0, nchunks, step=4)
                def _chunk(c):
                    for k in range(4):
                        sl = pl.ds((c + k) * 16, 16)
                        v = idxv[sl]
                        d0 = v.astype(jnp.int32)
                        d0f = d0.astype(jnp.float32)
                        w0 = v - d0f
                        w1 = jnp.float32(1.0) - w0
                        y0 = plsc.load_gather(trace, [d0])
                        y1 = plsc.load_gather(trace, [d0 + 1])
                        plsc.addupdate(acc_v.at[sl], w0 * y0 + w1 * y1)

        pltpu.sync_copy(acc_v, out_hbm.at[wid])

    return k(x, idxs)


def kernel(x, gridF, sensors):
    ax = gridF[:, 0, 0]
    ay = gridF[0, :, 1]
    sx = sensors[:, 0]
    du = ax[None, :] - sx[:, None]
    u2 = du * du
    dv = ay - sensors[0, 1]
    v2 = dv * dv
    ngrp = 8
    sg = _NS // ngrp
    parts = []
    for g in range(ngrp):
        u2_g = lax.slice_in_dim(u2, g * sg, (g + 1) * sg, axis=0)
        x_g = lax.slice_in_dim(x, g * sg, (g + 1) * sg, axis=1)
        idx_g = _compute_idx(u2_g, v2)
        idx_g = idx_g.reshape(sg, 32, _NX * _NY // 32)
        parts.append(_das_sc(x_g, idx_g, sg))
    out = functools.reduce(lambda a, b: a + b, parts)
    return out.reshape(1, _NX, _NY)


# speedup vs baseline: 2.1015x; 1.6852x over previous
import functools

import jax
import jax.numpy as jnp
import numpy as np
from jax import lax
from jax.experimental import pallas as pl
from jax.experimental.pallas import tpu as pltpu
from jax.experimental.pallas import tpu_sc as plsc

_NX, _NY, _NT, _NS = 512, 512, 2048, 256
_K = np.float32(1.0 / (1500.0 * 2e-8))
_TMAX_IDX = np.float32((_NT - 2) * 2e-8 / 2e-8)

_SBLK = 8
_IBLK = 128


def _idx_body(u2_ref, v2_ref, out_ref):
    u2 = u2_ref[...]
    v2 = v2_ref[...]
    d2 = u2[:, :, None] + v2[None, None, :]
    idx = jnp.sqrt(d2) * _K
    idx = jnp.where(idx > _TMAX_IDX, jnp.float32(0.0), idx)
    out_ref[...] = idx


def _compute_idx(u2, v2):
    ns = u2.shape[0]
    return pl.pallas_call(
        _idx_body,
        grid=(ns // _SBLK, _NX // _IBLK),
        in_specs=[
            pl.BlockSpec((_SBLK, _IBLK), lambda s, i: (s, i)),
            pl.BlockSpec((_NY,), lambda s, i: (0,)),
        ],
        out_specs=pl.BlockSpec((_SBLK, _IBLK, _NY), lambda s, i: (s, i, 0)),
        out_shape=jax.ShapeDtypeStruct((ns, _NX, _NY), jnp.float32),
    )(u2, v2)


def _das_sc(x, idxs, ns):
    info = plsc.get_sparse_core_info()
    nc, nsc = info.num_cores, info.num_subcores
    nw = nc * nsc
    px_w = _NX * _NY // nw
    nchunks = px_w // 16

    mesh = plsc.VectorSubcoreMesh(core_axis_name="c", subcore_axis_name="s")

    @functools.partial(
        pl.kernel,
        out_type=jax.ShapeDtypeStruct((nw, px_w), jnp.float32),
        mesh=mesh,
        scratch_types=[
            pltpu.VMEM((_NT,), jnp.float32),
            pltpu.VMEM((_NT,), jnp.float32),
            pltpu.VMEM((px_w,), jnp.float32),
            pltpu.VMEM((px_w,), jnp.float32),
            pltpu.VMEM((px_w,), jnp.float32),
            pltpu.SemaphoreType.DMA,
            pltpu.SemaphoreType.DMA,
            pltpu.SemaphoreType.DMA,
            pltpu.SemaphoreType.DMA,
        ],
        compiler_params=pltpu.CompilerParams(needs_layout_passes=False),
    )
    def k(x_hbm, idx_hbm, out_hbm, tr0, tr1, ix0, ix1, acc_v, st0, st1, si0, si1):
        wid = lax.axis_index("s") * nc + lax.axis_index("c")
        trace_b = [tr0, tr1]
        idx_b = [ix0, ix1]
        strc = [st0, st1]
        sidx = [si0, si1]

        def start(s, b):
            pltpu.async_copy(x_hbm.at[0, s], trace_b[b], strc[b])
            pltpu.async_copy(idx_hbm.at[s, wid], idx_b[b], sidx[b])

        def wait(s, b):
            pltpu.make_async_copy(x_hbm.at[0, s], trace_b[b], strc[b]).wait()
            pltpu.make_async_copy(idx_hbm.at[s, wid], idx_b[b], sidx[b]).wait()

        start(0, 0)

        zeros = jnp.zeros((16,), jnp.float32)

        @pl.loop(0, nchunks)
        def _z(c):
            acc_v[pl.ds(c * 16, 16)] = zeros

        @pl.loop(0, ns, step=2)
        def _sensor(s0):
            for b in range(2):
                s = s0 + b

                @pl.when(s + 1 < ns)
                def _prefetch():
                    start(s + 1, 1 - b)

                wait(s, b)
                trace = trace_b[b]
                idxv = idx_b[b]

                @plsc.parallel_loop(0, nchunks, unroll=4)
                def _chunk(c):
                    sl = pl.ds(c * 16, 16)
                    v = idxv[sl]
                    d0 = v.astype(jnp.int32)
                    d0f = d0.astype(jnp.float32)
                    w0 = v - d0f
                    w1 = jnp.float32(1.0) - w0
                    y0 = plsc.load_gather(trace, [d0])
                    y1 = plsc.load_gather(trace, [d0 + 1])
                    plsc.addupdate(acc_v.at[sl], w0 * y0 + w1 * y1)

        pltpu.sync_copy(acc_v, out_hbm.at[wid])

    return k(x, idxs)


def kernel(x, gridF, sensors):
    ax = gridF[:, 0, 0]
    ay = gridF[0, :, 1]
    sx = sensors[:, 0]
    du = ax[None, :] - sx[:, None]
    u2 = du * du
    dv = ay - sensors[0, 1]
    v2 = dv * dv
    ngrp = 8
    sg = _NS // ngrp
    parts = []
    for g in range(ngrp):
        u2_g = lax.slice_in_dim(u2, g * sg, (g + 1) * sg, axis=0)
        x_g = lax.slice_in_dim(x, g * sg, (g + 1) * sg, axis=1)
        idx_g = _compute_idx(u2_g, v2)
        idx_g = idx_g.reshape(sg, 32, _NX * _NY // 32)
        parts.append(_das_sc(x_g, idx_g, sg))
    out = functools.reduce(lambda a, b: a + b, parts)
    return out.reshape(1, _NX, _NY)
